# baseline (speedup 1.0000x reference)
import functools

import jax
import jax.numpy as jnp
from jax import lax
from jax.experimental import pallas as pl
from jax.experimental.pallas import tpu as pltpu
from jax.experimental.pallas import tpu_sc as plsc

N = 10000
E = 320000
D = 128
G = 16

NC, NS = 2, 16
NW = NC * NS
EP = E // NW
CH = 40
NCH = EP // CH
NBUF = 6
RSTRIDE = 624
RPT = 640

_MESH = plsc.VectorSubcoreMesh(
    core_axis_name="c", subcore_axis_name="s", num_cores=NC, num_subcores=NS)


@functools.partial(
    pl.kernel,
    out_type=jax.ShapeDtypeStruct((NC, N, 16), jnp.float32),
    mesh=_MESH,
    scratch_types=[
        pltpu.VMEM((NCH, CH), jnp.int32),
        pltpu.VMEM((CH, 16), jnp.float32),
        pltpu.VMEM_SHARED((N, 16), jnp.float32),
        pltpu.SemaphoreType.DMA,
    ],
    compiler_params=pltpu.CompilerParams(use_tc_tiling_on_sc=False),
)
def _sc_degree(dst_hbm, ones_hbm, zeros_hbm, out_hbm, dstv, onesv, acc, sem):
    cid = lax.axis_index("c")
    sid = lax.axis_index("s")
    wid = cid * NS + sid
    r0 = sid * RSTRIDE
    pltpu.sync_copy(dst_hbm.at[pl.ds(wid * NCH, NCH)], dstv)
    pltpu.sync_copy(zeros_hbm, acc.at[pl.ds(r0, RPT)])
    pltpu.sync_copy(ones_hbm, onesv)
    plsc.subcore_barrier()

    def fire(c, carry):
        pltpu.async_copy(onesv, acc.at[dstv.at[c]], sem, add=True)
        return carry

    lax.fori_loop(0, NCH, fire, 0)

    def drain(c, carry):
        pltpu.make_async_copy(onesv, acc.at[dstv.at[0]], sem).wait()
        return carry

    lax.fori_loop(0, NCH, drain, 0)
    plsc.subcore_barrier()
    pltpu.sync_copy(acc.at[pl.ds(r0, RPT)], out_hbm.at[cid, pl.ds(r0, RPT)])


@functools.partial(
    pl.kernel,
    out_type=jax.ShapeDtypeStruct((NC, N, D), jnp.float32),
    mesh=_MESH,
    scratch_types=(
        [pltpu.VMEM((NCH, CH), jnp.int32)] * 2
        + [pltpu.VMEM((CH, D), jnp.float32)] * NBUF
        + [pltpu.VMEM_SHARED((N, D), jnp.float32)]
        + [pltpu.SemaphoreType.DMA] * (2 * NBUF)
    ),
    compiler_params=pltpu.CompilerParams(use_tc_tiling_on_sc=False),
)
def _sc_propagate(hs_hbm, src_hbm, dst_hbm, zeros_hbm, out_hbm,
                  srcv, dstv, *ring):
    rows = ring[:NBUF]
    acc = ring[NBUF]
    gs = ring[NBUF + 1:2 * NBUF + 1]
    ss = ring[2 * NBUF + 1:]
    cid = lax.axis_index("c")
    sid = lax.axis_index("s")
    wid = cid * NS + sid
    r0 = sid * RSTRIDE
    c0 = wid * NCH
    pltpu.sync_copy(src_hbm.at[pl.ds(c0, NCH)], srcv)
    pltpu.sync_copy(dst_hbm.at[pl.ds(c0, NCH)], dstv)
    pltpu.sync_copy(zeros_hbm, acc.at[pl.ds(r0, RPT)])
    plsc.subcore_barrier()

    pltpu.async_copy(hs_hbm.at[srcv.at[0]], rows[0], gs[0])

    def body(c, carry):
        def stage(b, nb):
            pltpu.make_async_copy(hs_hbm.at[srcv.at[c]], rows[b],
                                  gs[b]).wait()
            pltpu.async_copy(rows[b], acc.at[dstv.at[c]], ss[b], add=True)

            @pl.when(c + 1 < NCH)
            def _():
                @pl.when(c >= NBUF - 1)
                def _():
                    pltpu.make_async_copy(rows[nb], acc.at[dstv.at[c]],
                                          ss[nb]).wait()
                pltpu.async_copy(hs_hbm.at[srcv.at[c + 1]], rows[nb], gs[nb])

        for b in range(NBUF):
            @pl.when(c % NBUF == b)
            def _(b=b):
                stage(b, (b + 1) % NBUF)

        return carry

    lax.fori_loop(0, NCH, body, 0)
    for b in range(NBUF):
        pltpu.make_async_copy(rows[b], acc.at[dstv.at[0]], ss[b]).wait()
    plsc.subcore_barrier()
    pltpu.sync_copy(acc.at[pl.ds(r0, RPT)], out_hbm.at[cid, pl.ds(r0, RPT)])


_BR = 1000
_GRID = N // _BR


def _dinv_mm_body(degp_ref, x_ref, w_ref, dinv_ref, hs_ref):
    deg = degp_ref[0, :, 0:1] + degp_ref[1, :, 0:1] + 1.0
    dinv = 1.0 / jnp.sqrt(deg)
    dinv_ref[...] = dinv
    hs_ref[...] = jnp.dot(x_ref[...], w_ref[...],
                          preferred_element_type=jnp.float32) * dinv


def _tc_dinv_mm(degp, x, w):
    row = pl.BlockSpec((_BR, D), lambda i: (i, 0))
    return pl.pallas_call(
        _dinv_mm_body,
        grid=(_GRID,),
        in_specs=[
            pl.BlockSpec((NC, _BR, 16), lambda i: (0, i, 0)),
            row,
            pl.BlockSpec((D, D), lambda i: (0, 0)),
        ],
        out_specs=[pl.BlockSpec((_BR, 1), lambda i: (i, 0)), row],
        out_shape=[jax.ShapeDtypeStruct((N, 1), jnp.float32),
                   jax.ShapeDtypeStruct((N, D), jnp.float32)],
    )(degp, x, w)


def _ln_post(has_res, relu, p_ref, hs_ref, dinv_ref, b_ref, g_ref, be_ref,
             res_ref):
    s = p_ref[0] + p_ref[1] + hs_ref[...]
    conv = dinv_ref[...] * s + b_ref[...]
    mu = jnp.mean(conv, axis=1, keepdims=True)
    xc = conv - mu
    var = jnp.mean(xc * xc, axis=1, keepdims=True)
    y = xc * lax.rsqrt(var + 1e-5) * g_ref[...] + be_ref[...]
    if has_res:
        y = y + res_ref[...]
    if relu:
        y = jnp.maximum(y, 0.0)
    return y


def _post_mm_body(has_res, relu, *refs):
    if has_res:
        (p_ref, hs_ref, dinv_ref, b_ref, g_ref, be_ref, res_ref, wn_ref,
         h_ref, hsn_ref) = refs
    else:
        (p_ref, hs_ref, dinv_ref, b_ref, g_ref, be_ref, wn_ref,
         h_ref, hsn_ref) = refs
        res_ref = None
    y = _ln_post(has_res, relu, p_ref, hs_ref, dinv_ref, b_ref, g_ref,
                 be_ref, res_ref)
    h_ref[...] = y
    hsn_ref[...] = jnp.dot(y, wn_ref[...],
                           preferred_element_type=jnp.float32) * dinv_ref[...]


def _tc_post_mm(p, hs, dinv, b, g, be, wn, res=None, relu=True):
    has_res = res is not None
    vec = pl.BlockSpec((1, D), lambda i: (0, 0))
    row = pl.BlockSpec((_BR, D), lambda i: (i, 0))
    in_specs = [
        pl.BlockSpec((NC, _BR, D), lambda i: (0, i, 0)),
        row,
        pl.BlockSpec((_BR, 1), lambda i: (i, 0)),
        vec, vec, vec,
    ]
    args = [p, hs, dinv, b.reshape(1, D), g.reshape(1, D), be.reshape(1, D)]
    if has_res:
        in_specs.append(row)
        args.append(res)
    in_specs.append(pl.BlockSpec((D, D), lambda i: (0, 0)))
    args.append(wn)
    return pl.pallas_call(
        functools.partial(_post_mm_body, has_res, relu),
        grid=(_GRID,),
        in_specs=in_specs,
        out_specs=[row, row],
        out_shape=[jax.ShapeDtypeStruct((N, D), jnp.float32),
                   jax.ShapeDtypeStruct((N, D), jnp.float32)],
    )(*args)


def _post_pool_body(p_ref, hs_ref, dinv_ref, b_ref, g_ref, be_ref, res_ref,
                    batch_ref, wl_ref, bl_ref, sums_ref, cnts_ref, out_ref):
    i = pl.program_id(0)

    @pl.when(i == 0)
    def _():
        sums_ref[...] = jnp.zeros_like(sums_ref)
        cnts_ref[...] = jnp.zeros_like(cnts_ref)

    y = _ln_post(True, False, p_ref, hs_ref, dinv_ref, b_ref, g_ref,
                 be_ref, res_ref)
    gids = lax.broadcasted_iota(jnp.int32, (_BR, G), 1)
    onehot = (batch_ref[...] == gids).astype(jnp.float32)
    dn = (((0,), (0,)), ((), ()))
    sums_ref[...] += lax.dot_general(
        onehot, y, dn, preferred_element_type=jnp.float32,
        precision=lax.Precision.HIGHEST)
    cnts_ref[...] += lax.dot_general(
        onehot, jnp.ones((_BR, D), jnp.float32), dn,
        preferred_element_type=jnp.float32)

    @pl.when(i == _GRID - 1)
    def _():
        mean = sums_ref[...] / jnp.maximum(cnts_ref[...], 1.0)
        out_ref[...] = jnp.dot(
            mean, wl_ref[...], preferred_element_type=jnp.float32) + bl_ref[...]


def _tc_post_pool(p, hs, dinv, b, g, be, res, batch2d, wl, bl2d):
    vec = pl.BlockSpec((1, D), lambda i: (0, 0))
    row = pl.BlockSpec((_BR, D), lambda i: (i, 0))
    acc = pl.BlockSpec((G, D), lambda i: (0, 0))
    _, _, out = pl.pallas_call(
        _post_pool_body,
        grid=(_GRID,),
        in_specs=[
            pl.BlockSpec((NC, _BR, D), lambda i: (0, i, 0)),
            row,
            pl.BlockSpec((_BR, 1), lambda i: (i, 0)),
            vec, vec, vec,
            row,
            pl.BlockSpec((_BR, 1), lambda i: (i, 0)),
            pl.BlockSpec((D, 1), lambda i: (0, 0)),
            pl.BlockSpec((1, 1), lambda i: (0, 0)),
        ],
        out_specs=[acc, acc, pl.BlockSpec((G, 1), lambda i: (0, 0))],
        out_shape=[
            jax.ShapeDtypeStruct((G, D), jnp.float32),
            jax.ShapeDtypeStruct((G, D), jnp.float32),
            jax.ShapeDtypeStruct((G, 1), jnp.float32),
        ],
    )(p, hs, dinv, b.reshape(1, D), g.reshape(1, D), be.reshape(1, D), res,
      batch2d, wl, bl2d)
    return out


def kernel(x, edge_index, batch, W0, b0, W1, b1, W2, b2,
           g0, be0, g1, be1, g2, be2, Wl, bl):
    src = edge_index[0].astype(jnp.int32)
    dst = edge_index[1].astype(jnp.int32)
    ones16 = jnp.ones((CH, 16), jnp.float32)
    zeros16 = jnp.zeros((RPT, 16), jnp.float32)
    zeros = jnp.zeros((RPT, D), jnp.float32)

    src2d = src.reshape(NW * NCH, CH)
    dst2d = dst.reshape(NW * NCH, CH)

    degp = _sc_degree(dst2d, ones16, zeros16)
    dinv, hs0 = _tc_dinv_mm(degp, x, W0)

    p0 = _sc_propagate(hs0, src2d, dst2d, zeros)
    h1, hs1 = _tc_post_mm(p0, hs0, dinv, b0, g0, be0, W1, res=None, relu=True)

    p1 = _sc_propagate(hs1, src2d, dst2d, zeros)
    h2, hs2 = _tc_post_mm(p1, hs1, dinv, b1, g1, be1, W2, res=h1, relu=True)

    p2 = _sc_propagate(hs2, src2d, dst2d, zeros)
    return _tc_post_pool(p2, hs2, dinv, b2, g2, be2, h2,
                         batch.astype(jnp.int32).reshape(N, 1), Wl,
                         bl.reshape(1, 1))


# speedup vs baseline: 1.3823x; 1.3823x over previous
import functools

import jax
import jax.numpy as jnp
from jax import lax
from jax.experimental import pallas as pl
from jax.experimental.pallas import tpu as pltpu
from jax.experimental.pallas import tpu_sc as plsc

N = 10000
E = 320000
D = 128
G = 16

NC, NS = 2, 16
NW = NC * NS
EP = E // NW
CH = 80
NCH = EP // CH
NBUF = 3
RSTRIDE = 624
RPT = 640

_MESH = plsc.VectorSubcoreMesh(
    core_axis_name="c", subcore_axis_name="s", num_cores=NC, num_subcores=NS)


@functools.partial(
    pl.kernel,
    out_type=jax.ShapeDtypeStruct((NC, N, 16), jnp.float32),
    mesh=_MESH,
    scratch_types=[
        pltpu.VMEM((NCH, CH), jnp.int32),
        pltpu.VMEM((CH, 16), jnp.float32),
        pltpu.VMEM_SHARED((N, 16), jnp.float32),
        pltpu.SemaphoreType.DMA,
    ],
    compiler_params=pltpu.CompilerParams(use_tc_tiling_on_sc=False),
)
def _sc_degree(dst_hbm, ones_hbm, zeros_hbm, out_hbm, dstv, onesv, acc, sem):
    cid = lax.axis_index("c")
    sid = lax.axis_index("s")
    wid = cid * NS + sid
    r0 = sid * RSTRIDE
    pltpu.sync_copy(dst_hbm.at[pl.ds(wid * NCH, NCH)], dstv)
    pltpu.sync_copy(zeros_hbm, acc.at[pl.ds(r0, RPT)])
    pltpu.sync_copy(ones_hbm, onesv)
    plsc.subcore_barrier()

    def fire(c, carry):
        pltpu.async_copy(onesv, acc.at[dstv.at[c]], sem, add=True)
        return carry

    lax.fori_loop(0, NCH, fire, 0)

    def drain(c, carry):
        pltpu.make_async_copy(onesv, acc.at[dstv.at[0]], sem).wait()
        return carry

    lax.fori_loop(0, NCH, drain, 0)
    plsc.subcore_barrier()
    pltpu.sync_copy(acc.at[pl.ds(r0, RPT)], out_hbm.at[cid, pl.ds(r0, RPT)])


@functools.partial(
    pl.kernel,
    out_type=jax.ShapeDtypeStruct((NC, N, D), jnp.float32),
    mesh=_MESH,
    scratch_types=(
        [pltpu.VMEM((NCH, CH), jnp.int32)] * 2
        + [pltpu.VMEM((CH, D), jnp.float32)] * NBUF
        + [pltpu.VMEM_SHARED((N, D), jnp.float32)]
        + [pltpu.SemaphoreType.DMA] * (2 * NBUF)
    ),
    compiler_params=pltpu.CompilerParams(use_tc_tiling_on_sc=False),
)
def _sc_propagate(hs_hbm, src_hbm, dst_hbm, zeros_hbm, out_hbm,
                  srcv, dstv, *ring):
    rows = ring[:NBUF]
    acc = ring[NBUF]
    gs = ring[NBUF + 1:2 * NBUF + 1]
    ss = ring[2 * NBUF + 1:]
    cid = lax.axis_index("c")
    sid = lax.axis_index("s")
    wid = cid * NS + sid
    r0 = sid * RSTRIDE
    c0 = wid * NCH
    pltpu.sync_copy(src_hbm.at[pl.ds(c0, NCH)], srcv)
    pltpu.sync_copy(dst_hbm.at[pl.ds(c0, NCH)], dstv)
    pltpu.sync_copy(zeros_hbm, acc.at[pl.ds(r0, RPT)])
    plsc.subcore_barrier()

    pltpu.async_copy(hs_hbm.at[srcv.at[0]], rows[0], gs[0])

    def body(c, carry):
        def stage(b, nb):
            pltpu.make_async_copy(hs_hbm.at[srcv.at[c]], rows[b],
                                  gs[b]).wait()
            pltpu.async_copy(rows[b], acc.at[dstv.at[c]], ss[b], add=True)

            @pl.when(c + 1 < NCH)
            def _():
                @pl.when(c >= NBUF - 1)
                def _():
                    pltpu.make_async_copy(rows[nb], acc.at[dstv.at[c]],
                                          ss[nb]).wait()
                pltpu.async_copy(hs_hbm.at[srcv.at[c + 1]], rows[nb], gs[nb])

        for b in range(NBUF):
            @pl.when(c % NBUF == b)
            def _(b=b):
                stage(b, (b + 1) % NBUF)

        return carry

    lax.fori_loop(0, NCH, body, 0)
    for b in range(NBUF):
        pltpu.make_async_copy(rows[b], acc.at[dstv.at[0]], ss[b]).wait()
    plsc.subcore_barrier()
    pltpu.sync_copy(acc.at[pl.ds(r0, RPT)], out_hbm.at[cid, pl.ds(r0, RPT)])


_BR = 1000
_GRID = N // _BR


def _dinv_mm_body(degp_ref, x_ref, w_ref, dinv_ref, hs_ref):
    deg = degp_ref[0, :, 0:1] + degp_ref[1, :, 0:1] + 1.0
    dinv = 1.0 / jnp.sqrt(deg)
    dinv_ref[...] = dinv
    hs_ref[...] = jnp.dot(x_ref[...], w_ref[...],
                          preferred_element_type=jnp.float32) * dinv


def _tc_dinv_mm(degp, x, w):
    row = pl.BlockSpec((_BR, D), lambda i: (i, 0))
    return pl.pallas_call(
        _dinv_mm_body,
        grid=(_GRID,),
        in_specs=[
            pl.BlockSpec((NC, _BR, 16), lambda i: (0, i, 0)),
            row,
            pl.BlockSpec((D, D), lambda i: (0, 0)),
        ],
        out_specs=[pl.BlockSpec((_BR, 1), lambda i: (i, 0)), row],
        out_shape=[jax.ShapeDtypeStruct((N, 1), jnp.float32),
                   jax.ShapeDtypeStruct((N, D), jnp.float32)],
    )(degp, x, w)


def _ln_post(has_res, relu, p_ref, hs_ref, dinv_ref, b_ref, g_ref, be_ref,
             res_ref):
    s = p_ref[0] + p_ref[1] + hs_ref[...]
    conv = dinv_ref[...] * s + b_ref[...]
    mu = jnp.mean(conv, axis=1, keepdims=True)
    xc = conv - mu
    var = jnp.mean(xc * xc, axis=1, keepdims=True)
    y = xc * lax.rsqrt(var + 1e-5) * g_ref[...] + be_ref[...]
    if has_res:
        y = y + res_ref[...]
    if relu:
        y = jnp.maximum(y, 0.0)
    return y


def _post_mm_body(has_res, relu, *refs):
    if has_res:
        (p_ref, hs_ref, dinv_ref, b_ref, g_ref, be_ref, res_ref, wn_ref,
         h_ref, hsn_ref) = refs
    else:
        (p_ref, hs_ref, dinv_ref, b_ref, g_ref, be_ref, wn_ref,
         h_ref, hsn_ref) = refs
        res_ref = None
    y = _ln_post(has_res, relu, p_ref, hs_ref, dinv_ref, b_ref, g_ref,
                 be_ref, res_ref)
    h_ref[...] = y
    hsn_ref[...] = jnp.dot(y, wn_ref[...],
                           preferred_element_type=jnp.float32) * dinv_ref[...]


def _tc_post_mm(p, hs, dinv, b, g, be, wn, res=None, relu=True):
    has_res = res is not None
    vec = pl.BlockSpec((1, D), lambda i: (0, 0))
    row = pl.BlockSpec((_BR, D), lambda i: (i, 0))
    in_specs = [
        pl.BlockSpec((NC, _BR, D), lambda i: (0, i, 0)),
        row,
        pl.BlockSpec((_BR, 1), lambda i: (i, 0)),
        vec, vec, vec,
    ]
    args = [p, hs, dinv, b.reshape(1, D), g.reshape(1, D), be.reshape(1, D)]
    if has_res:
        in_specs.append(row)
        args.append(res)
    in_specs.append(pl.BlockSpec((D, D), lambda i: (0, 0)))
    args.append(wn)
    return pl.pallas_call(
        functools.partial(_post_mm_body, has_res, relu),
        grid=(_GRID,),
        in_specs=in_specs,
        out_specs=[row, row],
        out_shape=[jax.ShapeDtypeStruct((N, D), jnp.float32),
                   jax.ShapeDtypeStruct((N, D), jnp.float32)],
    )(*args)


def _post_pool_body(p_ref, hs_ref, dinv_ref, b_ref, g_ref, be_ref, res_ref,
                    batch_ref, wl_ref, bl_ref, sums_ref, cnts_ref, out_ref):
    i = pl.program_id(0)

    @pl.when(i == 0)
    def _():
        sums_ref[...] = jnp.zeros_like(sums_ref)
        cnts_ref[...] = jnp.zeros_like(cnts_ref)

    y = _ln_post(True, False, p_ref, hs_ref, dinv_ref, b_ref, g_ref,
                 be_ref, res_ref)
    gids = lax.broadcasted_iota(jnp.int32, (_BR, G), 1)
    onehot = (batch_ref[...] == gids).astype(jnp.float32)
    dn = (((0,), (0,)), ((), ()))
    sums_ref[...] += lax.dot_general(
        onehot, y, dn, preferred_element_type=jnp.float32,
        precision=lax.Precision.HIGHEST)
    cnts_ref[...] += lax.dot_general(
        onehot, jnp.ones((_BR, D), jnp.float32), dn,
        preferred_element_type=jnp.float32)

    @pl.when(i == _GRID - 1)
    def _():
        mean = sums_ref[...] / jnp.maximum(cnts_ref[...], 1.0)
        out_ref[...] = jnp.dot(
            mean, wl_ref[...], preferred_element_type=jnp.float32) + bl_ref[...]


def _tc_post_pool(p, hs, dinv, b, g, be, res, batch2d, wl, bl2d):
    vec = pl.BlockSpec((1, D), lambda i: (0, 0))
    row = pl.BlockSpec((_BR, D), lambda i: (i, 0))
    acc = pl.BlockSpec((G, D), lambda i: (0, 0))
    _, _, out = pl.pallas_call(
        _post_pool_body,
        grid=(_GRID,),
        in_specs=[
            pl.BlockSpec((NC, _BR, D), lambda i: (0, i, 0)),
            row,
            pl.BlockSpec((_BR, 1), lambda i: (i, 0)),
            vec, vec, vec,
            row,
            pl.BlockSpec((_BR, 1), lambda i: (i, 0)),
            pl.BlockSpec((D, 1), lambda i: (0, 0)),
            pl.BlockSpec((1, 1), lambda i: (0, 0)),
        ],
        out_specs=[acc, acc, pl.BlockSpec((G, 1), lambda i: (0, 0))],
        out_shape=[
            jax.ShapeDtypeStruct((G, D), jnp.float32),
            jax.ShapeDtypeStruct((G, D), jnp.float32),
            jax.ShapeDtypeStruct((G, 1), jnp.float32),
        ],
    )(p, hs, dinv, b.reshape(1, D), g.reshape(1, D), be.reshape(1, D), res,
      batch2d, wl, bl2d)
    return out


def kernel(x, edge_index, batch, W0, b0, W1, b1, W2, b2,
           g0, be0, g1, be1, g2, be2, Wl, bl):
    src = edge_index[0].astype(jnp.int32)
    dst = edge_index[1].astype(jnp.int32)
    ones16 = jnp.ones((CH, 16), jnp.float32)
    zeros16 = jnp.zeros((RPT, 16), jnp.float32)
    zeros = jnp.zeros((RPT, D), jnp.float32)

    src2d = src.reshape(NW * NCH, CH)
    dst2d = dst.reshape(NW * NCH, CH)

    degp = _sc_degree(dst2d, ones16, zeros16)
    dinv, hs0 = _tc_dinv_mm(degp, x, W0)

    p0 = _sc_propagate(hs0, src2d, dst2d, zeros)
    h1, hs1 = _tc_post_mm(p0, hs0, dinv, b0, g0, be0, W1, res=None, relu=True)

    p1 = _sc_propagate(hs1, src2d, dst2d, zeros)
    h2, hs2 = _tc_post_mm(p1, hs1, dinv, b1, g1, be1, W2, res=h1, relu=True)

    p2 = _sc_propagate(hs2, src2d, dst2d, zeros)
    return _tc_post_pool(p2, hs2, dinv, b2, g2, be2, h2,
                         batch.astype(jnp.int32).reshape(N, 1), Wl,
                         bl.reshape(1, 1))


# speedup vs baseline: 1.3977x; 1.0111x over previous
import functools

import jax
import jax.numpy as jnp
from jax import lax
from jax.experimental import pallas as pl
from jax.experimental.pallas import tpu as pltpu
from jax.experimental.pallas import tpu_sc as plsc

N = 10000
E = 320000
D = 128
G = 16

NC, NS = 2, 16
NW = NC * NS
EP = E // NW
CH = 80
NCH = EP // CH
NBUF = 3
RSTRIDE = 624
RPT = 640

_MESH = plsc.VectorSubcoreMesh(
    core_axis_name="c", subcore_axis_name="s", num_cores=NC, num_subcores=NS)


@functools.partial(
    pl.kernel,
    out_type=jax.ShapeDtypeStruct((NC, N, 16), jnp.float32),
    mesh=_MESH,
    scratch_types=[
        pltpu.VMEM((NCH, CH), jnp.int32),
        pltpu.VMEM((CH, 16), jnp.float32),
        pltpu.VMEM_SHARED((N, 16), jnp.float32),
        pltpu.SemaphoreType.DMA,
    ],
    compiler_params=pltpu.CompilerParams(use_tc_tiling_on_sc=False),
)
def _sc_degree(dst_hbm, ones_hbm, zeros_hbm, out_hbm, dstv, onesv, acc, sem):
    cid = lax.axis_index("c")
    sid = lax.axis_index("s")
    wid = cid * NS + sid
    r0 = sid * RSTRIDE
    pltpu.sync_copy(dst_hbm.at[pl.ds(wid * NCH, NCH)], dstv)
    pltpu.sync_copy(zeros_hbm, acc.at[pl.ds(r0, RPT)])
    pltpu.sync_copy(ones_hbm, onesv)
    plsc.subcore_barrier()

    def fire(c, carry):
        pltpu.async_copy(onesv, acc.at[dstv.at[c]], sem, add=True)
        return carry

    lax.fori_loop(0, NCH, fire, 0)

    def drain(c, carry):
        pltpu.make_async_copy(onesv, acc.at[dstv.at[0]], sem).wait()
        return carry

    lax.fori_loop(0, NCH, drain, 0)
    plsc.subcore_barrier()
    pltpu.sync_copy(acc.at[pl.ds(r0, RPT)], out_hbm.at[cid, pl.ds(r0, RPT)])


@functools.partial(
    pl.kernel,
    out_type=jax.ShapeDtypeStruct((NC, N, D), jnp.float32),
    mesh=_MESH,
    scratch_types=(
        [pltpu.VMEM((NCH, CH), jnp.int32)] * 2
        + [pltpu.VMEM((CH, D), jnp.float32)] * NBUF
        + [pltpu.VMEM_SHARED((N, D), jnp.float32)]
        + [pltpu.SemaphoreType.DMA] * (2 * NBUF)
    ),
    compiler_params=pltpu.CompilerParams(use_tc_tiling_on_sc=False),
)
def _sc_propagate(hs_hbm, src_hbm, dst_hbm, zeros_hbm, out_hbm,
                  srcv, dstv, *ring):
    rows = ring[:NBUF]
    acc = ring[NBUF]
    gs = ring[NBUF + 1:2 * NBUF + 1]
    ss = ring[2 * NBUF + 1:]
    cid = lax.axis_index("c")
    sid = lax.axis_index("s")
    wid = cid * NS + sid
    r0 = sid * RSTRIDE
    c0 = wid * NCH
    pltpu.async_copy(src_hbm.at[pl.ds(c0, NCH)], srcv, gs[0])
    pltpu.async_copy(dst_hbm.at[pl.ds(c0, NCH)], dstv, gs[1])
    pltpu.async_copy(zeros_hbm, acc.at[pl.ds(r0, RPT)], ss[0])
    pltpu.make_async_copy(src_hbm.at[pl.ds(c0, NCH)], srcv, gs[0]).wait()
    pltpu.async_copy(hs_hbm.at[srcv.at[0]], rows[0], gs[0])
    pltpu.make_async_copy(dst_hbm.at[pl.ds(c0, NCH)], dstv, gs[1]).wait()
    pltpu.make_async_copy(zeros_hbm, acc.at[pl.ds(r0, RPT)], ss[0]).wait()
    plsc.subcore_barrier()

    def body(c, carry):
        def stage(b, nb):
            pltpu.make_async_copy(hs_hbm.at[srcv.at[c]], rows[b],
                                  gs[b]).wait()
            pltpu.async_copy(rows[b], acc.at[dstv.at[c]], ss[b], add=True)

            @pl.when(c + 1 < NCH)
            def _():
                @pl.when(c >= NBUF - 1)
                def _():
                    pltpu.make_async_copy(rows[nb], acc.at[dstv.at[c]],
                                          ss[nb]).wait()
                pltpu.async_copy(hs_hbm.at[srcv.at[c + 1]], rows[nb], gs[nb])

        for b in range(NBUF):
            @pl.when(c % NBUF == b)
            def _(b=b):
                stage(b, (b + 1) % NBUF)

        return carry

    lax.fori_loop(0, NCH, body, 0)
    for b in range(NBUF):
        pltpu.make_async_copy(rows[b], acc.at[dstv.at[0]], ss[b]).wait()
    plsc.subcore_barrier()
    pltpu.sync_copy(acc.at[pl.ds(r0, RPT)], out_hbm.at[cid, pl.ds(r0, RPT)])


_BR = 1000
_GRID = N // _BR


def _dinv_mm_body(degp_ref, x_ref, w_ref, dinv_ref, hs_ref):
    deg = degp_ref[0, :, 0:1] + degp_ref[1, :, 0:1] + 1.0
    dinv = 1.0 / jnp.sqrt(deg)
    dinv_ref[...] = dinv
    hs_ref[...] = jnp.dot(x_ref[...], w_ref[...],
                          preferred_element_type=jnp.float32) * dinv


def _tc_dinv_mm(degp, x, w):
    row = pl.BlockSpec((_BR, D), lambda i: (i, 0))
    return pl.pallas_call(
        _dinv_mm_body,
        grid=(_GRID,),
        in_specs=[
            pl.BlockSpec((NC, _BR, 16), lambda i: (0, i, 0)),
            row,
            pl.BlockSpec((D, D), lambda i: (0, 0)),
        ],
        out_specs=[pl.BlockSpec((_BR, 1), lambda i: (i, 0)), row],
        out_shape=[jax.ShapeDtypeStruct((N, 1), jnp.float32),
                   jax.ShapeDtypeStruct((N, D), jnp.float32)],
    )(degp, x, w)


def _ln_post(has_res, relu, p_ref, hs_ref, dinv_ref, b_ref, g_ref, be_ref,
             res_ref):
    s = p_ref[0] + p_ref[1] + hs_ref[...]
    conv = dinv_ref[...] * s + b_ref[...]
    mu = jnp.mean(conv, axis=1, keepdims=True)
    xc = conv - mu
    var = jnp.mean(xc * xc, axis=1, keepdims=True)
    y = xc * lax.rsqrt(var + 1e-5) * g_ref[...] + be_ref[...]
    if has_res:
        y = y + res_ref[...]
    if relu:
        y = jnp.maximum(y, 0.0)
    return y


def _post_mm_body(has_res, relu, *refs):
    if has_res:
        (p_ref, hs_ref, dinv_ref, b_ref, g_ref, be_ref, res_ref, wn_ref,
         h_ref, hsn_ref) = refs
    else:
        (p_ref, hs_ref, dinv_ref, b_ref, g_ref, be_ref, wn_ref,
         h_ref, hsn_ref) = refs
        res_ref = None
    y = _ln_post(has_res, relu, p_ref, hs_ref, dinv_ref, b_ref, g_ref,
                 be_ref, res_ref)
    h_ref[...] = y
    hsn_ref[...] = jnp.dot(y, wn_ref[...],
                           preferred_element_type=jnp.float32) * dinv_ref[...]


def _tc_post_mm(p, hs, dinv, b, g, be, wn, res=None, relu=True):
    has_res = res is not None
    vec = pl.BlockSpec((1, D), lambda i: (0, 0))
    row = pl.BlockSpec((_BR, D), lambda i: (i, 0))
    in_specs = [
        pl.BlockSpec((NC, _BR, D), lambda i: (0, i, 0)),
        row,
        pl.BlockSpec((_BR, 1), lambda i: (i, 0)),
        vec, vec, vec,
    ]
    args = [p, hs, dinv, b.reshape(1, D), g.reshape(1, D), be.reshape(1, D)]
    if has_res:
        in_specs.append(row)
        args.append(res)
    in_specs.append(pl.BlockSpec((D, D), lambda i: (0, 0)))
    args.append(wn)
    return pl.pallas_call(
        functools.partial(_post_mm_body, has_res, relu),
        grid=(_GRID,),
        in_specs=in_specs,
        out_specs=[row, row],
        out_shape=[jax.ShapeDtypeStruct((N, D), jnp.float32),
                   jax.ShapeDtypeStruct((N, D), jnp.float32)],
    )(*args)


def _post_pool_body(p_ref, hs_ref, dinv_ref, b_ref, g_ref, be_ref, res_ref,
                    batch_ref, wl_ref, bl_ref, sums_ref, cnts_ref, out_ref):
    i = pl.program_id(0)

    @pl.when(i == 0)
    def _():
        sums_ref[...] = jnp.zeros_like(sums_ref)
        cnts_ref[...] = jnp.zeros_like(cnts_ref)

    y = _ln_post(True, False, p_ref, hs_ref, dinv_ref, b_ref, g_ref,
                 be_ref, res_ref)
    gids = lax.broadcasted_iota(jnp.int32, (_BR, G), 1)
    onehot = (batch_ref[...] == gids).astype(jnp.float32)
    dn = (((0,), (0,)), ((), ()))
    sums_ref[...] += lax.dot_general(
        onehot, y, dn, preferred_element_type=jnp.float32,
        precision=lax.Precision.HIGHEST)
    cnts_ref[...] += lax.dot_general(
        onehot, jnp.ones((_BR, D), jnp.float32), dn,
        preferred_element_type=jnp.float32)

    @pl.when(i == _GRID - 1)
    def _():
        mean = sums_ref[...] / jnp.maximum(cnts_ref[...], 1.0)
        out_ref[...] = jnp.dot(
            mean, wl_ref[...], preferred_element_type=jnp.float32) + bl_ref[...]


def _tc_post_pool(p, hs, dinv, b, g, be, res, batch2d, wl, bl2d):
    vec = pl.BlockSpec((1, D), lambda i: (0, 0))
    row = pl.BlockSpec((_BR, D), lambda i: (i, 0))
    acc = pl.BlockSpec((G, D), lambda i: (0, 0))
    _, _, out = pl.pallas_call(
        _post_pool_body,
        grid=(_GRID,),
        in_specs=[
            pl.BlockSpec((NC, _BR, D), lambda i: (0, i, 0)),
            row,
            pl.BlockSpec((_BR, 1), lambda i: (i, 0)),
            vec, vec, vec,
            row,
            pl.BlockSpec((_BR, 1), lambda i: (i, 0)),
            pl.BlockSpec((D, 1), lambda i: (0, 0)),
            pl.BlockSpec((1, 1), lambda i: (0, 0)),
        ],
        out_specs=[acc, acc, pl.BlockSpec((G, 1), lambda i: (0, 0))],
        out_shape=[
            jax.ShapeDtypeStruct((G, D), jnp.float32),
            jax.ShapeDtypeStruct((G, D), jnp.float32),
            jax.ShapeDtypeStruct((G, 1), jnp.float32),
        ],
    )(p, hs, dinv, b.reshape(1, D), g.reshape(1, D), be.reshape(1, D), res,
      batch2d, wl, bl2d)
    return out


def kernel(x, edge_index, batch, W0, b0, W1, b1, W2, b2,
           g0, be0, g1, be1, g2, be2, Wl, bl):
    src = edge_index[0].astype(jnp.int32)
    dst = edge_index[1].astype(jnp.int32)
    ones16 = jnp.ones((CH, 16), jnp.float32)
    zeros16 = jnp.zeros((RPT, 16), jnp.float32)
    zeros = jnp.zeros((RPT, D), jnp.float32)

    src2d = src.reshape(NW * NCH, CH)
    dst2d = dst.reshape(NW * NCH, CH)

    degp = _sc_degree(dst2d, ones16, zeros16)
    dinv, hs0 = _tc_dinv_mm(degp, x, W0)

    p0 = _sc_propagate(hs0, src2d, dst2d, zeros)
    h1, hs1 = _tc_post_mm(p0, hs0, dinv, b0, g0, be0, W1, res=None, relu=True)

    p1 = _sc_propagate(hs1, src2d, dst2d, zeros)
    h2, hs2 = _tc_post_mm(p1, hs1, dinv, b1, g1, be1, W2, res=h1, relu=True)

    p2 = _sc_propagate(hs2, src2d, dst2d, zeros)
    return _tc_post_pool(p2, hs2, dinv, b2, g2, be2, h2,
                         batch.astype(jnp.int32).reshape(N, 1), Wl,
                         bl.reshape(1, 1))
